# baseline (speedup 1.0000x reference)
import functools

import jax
import jax.numpy as jnp
from jax import lax
from jax.experimental import pallas as pl
from jax.experimental.pallas import tpu as pltpu
from jax.experimental.pallas import tpu_sc as plsc

N = 10000
E = 320000
D = 128

NC = 2
NS = 16
NW = NC * NS
EPW = E // NW
CHUNK = 80
NCHUNK = EPW // CHUNK
NBUF = 4
GRP = 8
NPAD = 10240
RPT = NPAD // NS
ZROWS = 32
DEGW = 128

_mesh = plsc.VectorSubcoreMesh(core_axis_name="c", subcore_axis_name="s")


@functools.partial(
    pl.kernel,
    out_type=jax.ShapeDtypeStruct((NC * NPAD, DEGW), jnp.float32),
    mesh=_mesh,
    scratch_types=[
        [pltpu.VMEM((CHUNK,), jnp.int32)] * NBUF,
        pltpu.VMEM((CHUNK, DEGW), jnp.float32),
        pltpu.VMEM((ZROWS, DEGW), jnp.float32),
        pltpu.VMEM_SHARED((NPAD, DEGW), jnp.float32),
        [pltpu.SemaphoreType.DMA] * NBUF,
        [pltpu.SemaphoreType.DMA] * NBUF,
    ],
)
def _sc_degree(dst_hbm, ones_hbm, zeros_hbm, out_hbm, dbufs, ones_v, zbuf,
               acc_sh, isems, ssems):
    c = lax.axis_index("c")
    s = lax.axis_index("s")
    wid = s * NC + c
    ebase = wid * EPW

    pltpu.sync_copy(ones_hbm, ones_v)
    pltpu.sync_copy(zeros_hbm, zbuf)

    base_row = s * RPT
    for j in range(RPT // ZROWS):
        pltpu.sync_copy(zbuf, acc_sh.at[pl.ds(base_row + j * ZROWS, ZROWS)])

    def i_issue(cc, b):
        off = ebase + cc * CHUNK
        return pltpu.async_copy(dst_hbm.at[pl.ds(off, CHUNK)], dbufs[b],
                                isems[b])

    def s_issue(b):
        return pltpu.async_copy(ones_v, acc_sh.at[dbufs[b]], ssems[b],
                                add=True)

    plsc.subcore_barrier()

    def body(g, carry):
        base = g * GRP
        idesc, sdesc = {}, {}
        for b in range(NBUF):
            idesc[b] = i_issue(base + b, b)
        for q in range(GRP // NBUF):
            for b in range(NBUF):
                j = q * NBUF + b
                idesc[j].wait()
                sdesc[j] = s_issue(b)
            for b in range(NBUF):
                j = q * NBUF + b
                sdesc[j].wait()
                if j + NBUF < GRP:
                    idesc[j + NBUF] = i_issue(base + j + NBUF, b)
        return carry

    lax.fori_loop(0, NCHUNK // GRP, body, 0)
    for cc in range(NCHUNK - NCHUNK % GRP, NCHUNK):
        i_issue(cc, 0).wait()
        s_issue(0).wait()
    plsc.subcore_barrier()
    pltpu.sync_copy(acc_sh.at[pl.ds(base_row, RPT)],
                    out_hbm.at[pl.ds(c * NPAD + base_row, RPT)])


@functools.partial(
    pl.kernel,
    out_type=jax.ShapeDtypeStruct((NC * NPAD, D), jnp.float32),
    mesh=_mesh,
    scratch_types=[
        [pltpu.VMEM((CHUNK,), jnp.int32)] * NBUF,
        [pltpu.VMEM((CHUNK,), jnp.int32)] * NBUF,
        [pltpu.VMEM((CHUNK, D), jnp.float32)] * NBUF,
        pltpu.VMEM((ZROWS, D), jnp.float32),
        pltpu.VMEM_SHARED((NPAD, D), jnp.float32),
        [pltpu.SemaphoreType.DMA] * NBUF,
        [pltpu.SemaphoreType.DMA] * NBUF,
        [pltpu.SemaphoreType.DMA] * NBUF,
    ],
)
def _sc_agg(y_hbm, src_hbm, dst_hbm, out_hbm, sbufs, dbufs, rbufs, zbuf,
            acc_sh, isems, gsems, ssems):
    c = lax.axis_index("c")
    s = lax.axis_index("s")
    wid = s * NC + c
    ebase = wid * EPW

    def fill_zeros(r, carry):
        for k in range(D // 16):
            zbuf[r, pl.ds(k * 16, 16)] = jnp.zeros((16,), jnp.float32)
        return carry

    lax.fori_loop(0, ZROWS, fill_zeros, 0)

    base_row = s * RPT
    for j in range(RPT // ZROWS):
        pltpu.sync_copy(zbuf, acc_sh.at[pl.ds(base_row + j * ZROWS, ZROWS)])

    def i_issue(cc, b):
        off = ebase + cc * CHUNK
        return (
            pltpu.async_copy(src_hbm.at[pl.ds(off, CHUNK)], sbufs[b],
                             isems[b]),
            pltpu.async_copy(dst_hbm.at[pl.ds(off, CHUNK)], dbufs[b],
                             isems[b]),
        )

    def g_issue(b):
        return pltpu.async_copy(y_hbm.at[sbufs[b]], rbufs[b], gsems[b])

    def s_issue(b):
        return pltpu.async_copy(rbufs[b], acc_sh.at[dbufs[b]], ssems[b],
                                add=True)

    plsc.subcore_barrier()

    def body(g, carry):
        base = g * GRP
        idesc, gdesc, sdesc = {}, {}, {}
        for b in range(NBUF):
            idesc[b] = i_issue(base + b, b)
        for q in range(GRP // NBUF):
            for b in range(NBUF):
                j = q * NBUF + b
                for d in idesc[j]:
                    d.wait()
                gdesc[j] = g_issue(b)
            for b in range(NBUF):
                j = q * NBUF + b
                gdesc[j].wait()
                sdesc[j] = s_issue(b)
            for b in range(NBUF):
                j = q * NBUF + b
                sdesc[j].wait()
                if j + NBUF < GRP:
                    idesc[j + NBUF] = i_issue(base + j + NBUF, b)
        return carry

    lax.fori_loop(0, NCHUNK // GRP, body, 0)
    for cc in range(NCHUNK - NCHUNK % GRP, NCHUNK):
        for d in i_issue(cc, 0):
            d.wait()
        g_issue(0).wait()
        s_issue(0).wait()
    plsc.subcore_barrier()
    pltpu.sync_copy(acc_sh.at[pl.ds(base_row, RPT)],
                    out_hbm.at[pl.ds(c * NPAD + base_row, RPT)])


_R = 1000


def _tc_mm_body(x_ref, w_ref, h_ref):
    h_ref[...] = jnp.dot(x_ref[...], w_ref[...],
                         preferred_element_type=jnp.float32)


def _tc_mm(x, W1):
    return pl.pallas_call(
        _tc_mm_body,
        grid=(N // _R,),
        in_specs=[
            pl.BlockSpec((_R, D), lambda i: (i, 0)),
            pl.BlockSpec((D, D), lambda i: (0, 0)),
        ],
        out_specs=pl.BlockSpec((_R, D), lambda i: (i, 0)),
        out_shape=jax.ShapeDtypeStruct((N, D), jnp.float32),
    )(x, W1)


def _tc1_body(degp_ref, h_ref, y_ref, dinv_ref):
    deg = degp_ref[0, :, :1] + degp_ref[1, :, :1] + 1.0
    dinv = lax.rsqrt(deg)
    y_ref[...] = h_ref[...] * dinv
    dinv_ref[...] = jnp.broadcast_to(dinv, (_R, DEGW))


def _tc1(degp, h):
    return pl.pallas_call(
        _tc1_body,
        grid=(N // _R,),
        in_specs=[
            pl.BlockSpec((NC, _R, DEGW), lambda i: (0, i, 0)),
            pl.BlockSpec((_R, D), lambda i: (i, 0)),
        ],
        out_specs=[
            pl.BlockSpec((_R, D), lambda i: (i, 0)),
            pl.BlockSpec((_R, DEGW), lambda i: (i, 0)),
        ],
        out_shape=[
            jax.ShapeDtypeStruct((N, D), jnp.float32),
            jax.ShapeDtypeStruct((N, DEGW), jnp.float32),
        ],
    )(degp, h)


def _tc2_body(p_ref, y1_ref, dinv_ref, b1_ref, w2_ref, y2_ref):
    dinv = dinv_ref[...][:, :1]
    agg = p_ref[0] + p_ref[1] + y1_ref[...]
    z = jnp.maximum(agg * dinv + b1_ref[...], 0.0)
    y2_ref[...] = jnp.dot(z, w2_ref[...],
                          preferred_element_type=jnp.float32) * dinv


def _tc2(p1, y1, dinv16, b1, W2):
    return pl.pallas_call(
        _tc2_body,
        grid=(N // _R,),
        in_specs=[
            pl.BlockSpec((NC, _R, D), lambda i: (0, i, 0)),
            pl.BlockSpec((_R, D), lambda i: (i, 0)),
            pl.BlockSpec((_R, DEGW), lambda i: (i, 0)),
            pl.BlockSpec((1, D), lambda i: (0, 0)),
            pl.BlockSpec((D, D), lambda i: (0, 0)),
        ],
        out_specs=pl.BlockSpec((_R, D), lambda i: (i, 0)),
        out_shape=jax.ShapeDtypeStruct((N, D), jnp.float32),
    )(p1, y1, dinv16, b1, W2)


def _tc3_body(p_ref, y2_ref, dinv_ref, b2_ref, out_ref):
    dinv = dinv_ref[...][:, :1]
    out_ref[...] = (p_ref[0] + p_ref[1] + y2_ref[...]) * dinv + b2_ref[...]


def _tc3(p2, y2, dinv16, b2):
    return pl.pallas_call(
        _tc3_body,
        grid=(N // _R,),
        in_specs=[
            pl.BlockSpec((NC, _R, D), lambda i: (0, i, 0)),
            pl.BlockSpec((_R, D), lambda i: (i, 0)),
            pl.BlockSpec((_R, DEGW), lambda i: (i, 0)),
            pl.BlockSpec((1, D), lambda i: (0, 0)),
        ],
        out_specs=pl.BlockSpec((_R, D), lambda i: (i, 0)),
        out_shape=jax.ShapeDtypeStruct((N, D), jnp.float32),
    )(p2, y2, dinv16, b2)


def kernel(x, edge_index, W1, b1, W2, b2):
    src = edge_index[0]
    dst = edge_index[1]
    ones_c = jnp.ones((CHUNK, DEGW), jnp.float32)
    zeros_c = jnp.zeros((ZROWS, DEGW), jnp.float32)
    degp = _sc_degree(dst, ones_c, zeros_c).reshape(NC, NPAD, DEGW)
    h1 = _tc_mm(x, W1)
    y1, dinv16 = _tc1(degp, h1)
    p1 = _sc_agg(y1, src, dst).reshape(NC, NPAD, D)
    y2 = _tc2(p1, y1, dinv16, b1.reshape(1, D), W2)
    p2 = _sc_agg(y2, src, dst).reshape(NC, NPAD, D)
    return _tc3(p2, y2, dinv16, b2.reshape(1, D))
